# baseline (speedup 1.0000x reference)
import functools

import jax
import jax.numpy as jnp
from jax import lax
from jax.experimental import pallas as pl
from jax.experimental.pallas import tpu as pltpu
from jax.experimental.pallas import tpu_sc as plsc

N = 10000
NPAD = 10240
E = 320000
F = 128
NC, NS = 2, 16
EPS = E // NS
CH = 125
NSEG = 5
SCHUNK = 32
NPAIR = SCHUNK // 2
NH = NPAD // NC
HTRASH = NH
HROWS = 5248
DEGW = 16
DDEPTH = 8

_mesh = plsc.VectorSubcoreMesh(
    core_axis_name="c", subcore_axis_name="s", num_cores=NC, num_subcores=NS
)


def _map_dst(idx_in, idx_out, base, idx_src=None):
    offs = [min(k * 16, CH - 16) for k in range((CH + 15) // 16)]

    def body(j, carry):
        for o in offs:
            v = idx_in[j, pl.ds(o, 16)] - base
            m = (v >= 0) & (v < NH)
            idx_out[j, pl.ds(o, 16)] = jnp.where(m, v, HTRASH + (v & 127))
            if idx_src is not None:
                vs = idx_src[j, pl.ds(o, 16)]
                idx_src[j, pl.ds(o, 16)] = jnp.where(m, vs, 0)
        return carry

    lax.fori_loop(0, SCHUNK, body, 0)


def _fill_rows(ref, nrows, ncols, value):
    vec = jnp.full((16,), value, jnp.float32)

    def body(i, carry):
        for k in range(ncols // 16):
            ref[i, pl.ds(k * 16, 16)] = vec
        return carry

    lax.fori_loop(0, nrows, body, 0)


def _zero_shared(zbuf, sh, s):
    _fill_rows(zbuf, CH, zbuf.shape[1], 0.0)
    base = s * (HROWS // NS)
    pltpu.sync_copy(zbuf.at[pl.ds(0, 120)], sh.at[pl.ds(base, 120)])
    pltpu.sync_copy(zbuf.at[pl.ds(0, 120)], sh.at[pl.ds(base + 120, 120)])
    pltpu.sync_copy(zbuf.at[pl.ds(0, 88)], sh.at[pl.ds(base + 240, 88)])


def _copy_out(buf, sh, out_hbm, c, s):
    for k in range(4):
        r0 = s * 320 + k * 80
        pltpu.sync_copy(sh.at[pl.ds(r0, 80)], buf.at[pl.ds(0, 80)])
        pltpu.sync_copy(buf.at[pl.ds(0, 80)],
                        out_hbm.at[pl.ds(c * NH + r0, 80)])


@functools.partial(
    pl.kernel,
    out_type=jax.ShapeDtypeStruct((NPAD, DEGW), jnp.float32),
    mesh=_mesh,
    scratch_types=[
        pltpu.VMEM((SCHUNK, CH), jnp.int32),
        pltpu.VMEM((SCHUNK, CH), jnp.int32),
        pltpu.VMEM((CH, DEGW), jnp.float32),
        pltpu.VMEM((CH, DEGW), jnp.float32),
        pltpu.VMEM_SHARED((HROWS, DEGW), jnp.float32),
        pltpu.SemaphoreType.DMA,
    ],
)
def _deg_kernel(dst_hbm, out_hbm, idx_d, idx_dm, ones_v, buf, deg_sh, sem):
    c = lax.axis_index("c")
    s = lax.axis_index("s")
    _fill_rows(ones_v, CH, DEGW, 1.0)
    _zero_shared(buf, deg_sh, s)
    plsc.subcore_barrier()

    def seg(g, carry):
        pltpu.sync_copy(dst_hbm.at[s, g], idx_d)
        _map_dst(idx_d, idx_dm, c * NH)

        def body(j, carry2):
            pltpu.async_copy(ones_v, deg_sh.at[idx_dm.at[j]], sem, add=True)

            @pl.when(j >= DDEPTH)
            def _():
                pltpu.make_async_copy(
                    out_hbm.at[idx_dm.at[0]], ones_v, sem).wait()

            return carry2

        lax.fori_loop(0, SCHUNK, body, 0)

        def drain(j, carry2):
            pltpu.make_async_copy(
                out_hbm.at[idx_dm.at[0]], ones_v, sem).wait()
            return carry2

        lax.fori_loop(0, DDEPTH, drain, 0)
        return carry

    lax.fori_loop(0, NSEG, seg, 0)
    plsc.subcore_barrier()
    _copy_out(buf, deg_sh, out_hbm, c, s)


@functools.partial(
    pl.kernel,
    out_type=jax.ShapeDtypeStruct((NPAD, F), jnp.float32),
    mesh=_mesh,
    scratch_types=[
        pltpu.VMEM((SCHUNK, CH), jnp.int32),
        pltpu.VMEM((SCHUNK, CH), jnp.int32),
        pltpu.VMEM((SCHUNK, CH), jnp.int32),
        pltpu.VMEM((CH, F), jnp.float32),
        pltpu.VMEM((CH, F), jnp.float32),
        pltpu.VMEM_SHARED((HROWS, F), jnp.float32),
        pltpu.SemaphoreType.DMA,
        pltpu.SemaphoreType.DMA,
        pltpu.SemaphoreType.DMA,
        pltpu.SemaphoreType.DMA,
    ],
)
def _scatter_kernel(hs_hbm, src_hbm, dst_hbm, out_hbm,
                    idx_s, idx_d, idx_dm, r0, r1, acc_sh,
                    semg0, semg1, sems0, sems1):
    c = lax.axis_index("c")
    s = lax.axis_index("s")
    _zero_shared(r0, acc_sh, s)
    plsc.subcore_barrier()

    def drain(dst, sem):
        pltpu.make_async_copy(hs_hbm.at[idx_s.at[0]], dst, sem).wait()

    def seg(g, carry):
        pltpu.sync_copy(src_hbm.at[s, g], idx_s)
        pltpu.sync_copy(dst_hbm.at[s, g], idx_d)
        _map_dst(idx_d, idx_dm, c * NH, idx_src=idx_s)
        pltpu.async_copy(hs_hbm.at[idx_s.at[0]], r0, semg0)

        def pair(i, carry2):
            pltpu.async_copy(hs_hbm.at[idx_s.at[2 * i + 1]], r1, semg1)
            drain(r0, semg0)
            pltpu.async_copy(r0, acc_sh.at[idx_dm.at[2 * i]], sems0,
                             add=True)

            @pl.when(i < NPAIR - 1)
            def _():
                drain(r0, sems0)
                pltpu.async_copy(hs_hbm.at[idx_s.at[2 * i + 2]], r0, semg0)

            drain(r1, semg1)
            pltpu.async_copy(r1, acc_sh.at[idx_dm.at[2 * i + 1]], sems1,
                             add=True)

            @pl.when(i < NPAIR - 1)
            def _():
                drain(r1, sems1)

            return carry2

        lax.fori_loop(0, NPAIR, pair, 0)
        drain(r0, sems0)
        drain(r1, sems1)
        return carry

    lax.fori_loop(0, NSEG, seg, 0)
    plsc.subcore_barrier()
    _copy_out(r0, acc_sh, out_hbm, c, s)


BR = 1024
GRID = NPAD // BR

_DEGP_SPEC = pl.BlockSpec((BR, DEGW), lambda i: (i, 0))
_DIS_SPEC = pl.BlockSpec((BR, 16), lambda i: (i, 0))
_ROW_SPEC = pl.BlockSpec((BR, F), lambda i: (i, 0))
_MAT_SPEC = pl.BlockSpec((F, F), lambda i: (0, 0))
_VEC_SPEC = pl.BlockSpec((1, F), lambda i: (0, 0))


def _tcdis_body(degp_ref, dis_ref):
    deg = degp_ref[:, 0] + 1.0
    dis_ref[...] = jnp.broadcast_to(lax.rsqrt(deg)[:, None], (BR, 16))


_tcdis = pl.pallas_call(
    _tcdis_body,
    grid=(GRID,),
    in_specs=[_DEGP_SPEC],
    out_specs=_DIS_SPEC,
    out_shape=jax.ShapeDtypeStruct((NPAD, 16), jnp.float32),
)


def _tca_body(dis_ref, z_ref, w_ref, hs_ref):
    h = jnp.dot(z_ref[...], w_ref[...], preferred_element_type=jnp.float32)
    hs_ref[...] = dis_ref[:, :1] * h


_tca = pl.pallas_call(
    _tca_body,
    grid=(GRID,),
    in_specs=[_DIS_SPEC, _ROW_SPEC, _MAT_SPEC],
    out_specs=_ROW_SPEC,
    out_shape=jax.ShapeDtypeStruct((N, F), jnp.float32),
)


def _tcb_body(dis_ref, acc_ref, hs_ref, b_ref, zpre_ref, zrelu_ref):
    zpre = dis_ref[:, :1] * (acc_ref[...] + hs_ref[...]) + b_ref[...]
    zpre_ref[...] = zpre
    zrelu_ref[...] = jnp.maximum(zpre, 0.0)


_tcb = pl.pallas_call(
    _tcb_body,
    grid=(GRID,),
    in_specs=[_DIS_SPEC, _ROW_SPEC, _ROW_SPEC, _VEC_SPEC],
    out_specs=[_ROW_SPEC, _ROW_SPEC],
    out_shape=[jax.ShapeDtypeStruct((N, F), jnp.float32),
               jax.ShapeDtypeStruct((N, F), jnp.float32)],
)


def kernel(x, edge_index, W1, b1, W2, b2):
    src = edge_index[0].astype(jnp.int32).reshape(NS, NSEG, SCHUNK, CH)
    dst = edge_index[1].astype(jnp.int32).reshape(NS, NSEG, SCHUNK, CH)
    degp = _deg_kernel(dst)
    dis = _tcdis(degp)
    hs1 = _tca(dis, x, W1)
    acc1 = _scatter_kernel(hs1, src, dst)
    _, z2 = _tcb(dis, acc1, hs1, b1.reshape(1, F))
    hs2 = _tca(dis, z2, W2)
    acc2 = _scatter_kernel(hs2, src, dst)
    out, _ = _tcb(dis, acc2, hs2, b2.reshape(1, F))
    return out


# speedup vs baseline: 49.5375x; 49.5375x over previous
import functools

import jax
import jax.numpy as jnp
from jax import lax
from jax.experimental import pallas as pl
from jax.experimental.pallas import tpu as pltpu
from jax.experimental.pallas import tpu_sc as plsc

N = 10000
NPAD = 10240
E = 320000
F = 128
NC, NS = 2, 16
EPS = E // NS
CH = 125
NSEG = 5
SCHUNK = 32
NPAIR = SCHUNK // 2
NH = NPAD // NC
HTRASH = NH
HROWS = 5248
DEGW = 16
DDEPTH = 8

_mesh = plsc.VectorSubcoreMesh(
    core_axis_name="c", subcore_axis_name="s", num_cores=NC, num_subcores=NS
)


def _map_dst(idx_in, idx_out, base, idx_src=None):
    offs = [min(k * 16, CH - 16) for k in range((CH + 15) // 16)]

    def body(j, carry):
        for o in offs:
            v = idx_in[j, pl.ds(o, 16)] - base
            m = (v >= 0) & (v < NH)
            idx_out[j, pl.ds(o, 16)] = jnp.where(m, v, HTRASH + (v & 127))
            if idx_src is not None:
                vs = idx_src[j, pl.ds(o, 16)]
                idx_src[j, pl.ds(o, 16)] = jnp.where(m, vs, 0)
        return carry

    lax.fori_loop(0, SCHUNK, body, 0)


def _fill_rows(ref, nrows, ncols, value):
    vec = jnp.full((16,), value, jnp.float32)

    def body(i, carry):
        for k in range(ncols // 16):
            ref[i, pl.ds(k * 16, 16)] = vec
        return carry

    lax.fori_loop(0, nrows, body, 0)


def _zero_shared(zbuf, sh, s):
    _fill_rows(zbuf, CH, zbuf.shape[1], 0.0)
    base = s * (HROWS // NS)
    pltpu.sync_copy(zbuf.at[pl.ds(0, 120)], sh.at[pl.ds(base, 120)])
    pltpu.sync_copy(zbuf.at[pl.ds(0, 120)], sh.at[pl.ds(base + 120, 120)])
    pltpu.sync_copy(zbuf.at[pl.ds(0, 88)], sh.at[pl.ds(base + 240, 88)])


def _copy_out(buf, sh, out_hbm, c, s):
    for k in range(4):
        r0 = s * 320 + k * 80
        pltpu.sync_copy(sh.at[pl.ds(r0, 80)], buf.at[pl.ds(0, 80)])
        pltpu.sync_copy(buf.at[pl.ds(0, 80)],
                        out_hbm.at[pl.ds(c * NH + r0, 80)])


@functools.partial(
    pl.kernel,
    out_type=jax.ShapeDtypeStruct((NPAD, DEGW), jnp.float32),
    mesh=_mesh,
    scratch_types=[
        pltpu.VMEM((SCHUNK, CH), jnp.int32),
        pltpu.VMEM((SCHUNK, CH), jnp.int32),
        pltpu.VMEM((CH, DEGW), jnp.float32),
        pltpu.VMEM((CH, DEGW), jnp.float32),
        pltpu.VMEM_SHARED((HROWS, DEGW), jnp.float32),
        pltpu.SemaphoreType.DMA,
    ],
)
def _deg_kernel(dst_hbm, out_hbm, idx_d, idx_dm, ones_v, buf, deg_sh, sem):
    c = lax.axis_index("c")
    s = lax.axis_index("s")
    _fill_rows(ones_v, CH, DEGW, 1.0)
    _zero_shared(buf, deg_sh, s)
    plsc.subcore_barrier()

    def seg(g, carry):
        pltpu.sync_copy(dst_hbm.at[s, g], idx_d)
        _map_dst(idx_d, idx_dm, c * NH)

        def body(j, carry2):
            pltpu.async_copy(ones_v, deg_sh.at[idx_dm.at[j]], sem, add=True)

            @pl.when(j >= DDEPTH)
            def _():
                pltpu.make_async_copy(
                    out_hbm.at[idx_dm.at[0]], ones_v, sem).wait()

            return carry2

        lax.fori_loop(0, SCHUNK, body, 0)

        def drain(j, carry2):
            pltpu.make_async_copy(
                out_hbm.at[idx_dm.at[0]], ones_v, sem).wait()
            return carry2

        lax.fori_loop(0, DDEPTH, drain, 0)
        return carry

    lax.fori_loop(0, NSEG, seg, 0)
    plsc.subcore_barrier()
    _copy_out(buf, deg_sh, out_hbm, c, s)


@functools.partial(
    pl.kernel,
    out_type=jax.ShapeDtypeStruct((NPAD, F), jnp.float32),
    mesh=_mesh,
    scratch_types=[
        pltpu.VMEM((SCHUNK, CH), jnp.int32),
        pltpu.VMEM((SCHUNK, CH), jnp.int32),
        pltpu.VMEM((SCHUNK, CH), jnp.int32),
        pltpu.VMEM((CH, F), jnp.float32),
        pltpu.VMEM((CH, F), jnp.float32),
        pltpu.VMEM_SHARED((HROWS, F), jnp.float32),
        pltpu.SemaphoreType.DMA,
        pltpu.SemaphoreType.DMA,
        pltpu.SemaphoreType.DMA,
        pltpu.SemaphoreType.DMA,
    ],
)
def _scatter_kernel(hs_hbm, src_hbm, dst_hbm, out_hbm,
                    idx_s, idx_d, idx_dm, r0, r1, acc_sh,
                    semg0, semg1, sems0, sems1):
    c = lax.axis_index("c")
    s = lax.axis_index("s")
    _zero_shared(r0, acc_sh, s)
    plsc.subcore_barrier()

    def drain(dst, sem):
        pltpu.make_async_copy(hs_hbm.at[idx_s.at[0]], dst, sem).wait()

    def seg(g, carry):
        pltpu.sync_copy(src_hbm.at[s, g], idx_s)
        pltpu.sync_copy(dst_hbm.at[s, g], idx_d)
        _map_dst(idx_d, idx_dm, c * NH)
        pltpu.async_copy(hs_hbm.at[idx_s.at[0]], r0, semg0)

        def pair(i, carry2):
            pltpu.async_copy(hs_hbm.at[idx_s.at[2 * i + 1]], r1, semg1)
            drain(r0, semg0)
            pltpu.async_copy(r0, acc_sh.at[idx_dm.at[2 * i]], sems0,
                             add=True)

            @pl.when(i < NPAIR - 1)
            def _():
                drain(r0, sems0)
                pltpu.async_copy(hs_hbm.at[idx_s.at[2 * i + 2]], r0, semg0)

            drain(r1, semg1)
            pltpu.async_copy(r1, acc_sh.at[idx_dm.at[2 * i + 1]], sems1,
                             add=True)

            @pl.when(i < NPAIR - 1)
            def _():
                drain(r1, sems1)

            return carry2

        lax.fori_loop(0, NPAIR, pair, 0)
        drain(r0, sems0)
        drain(r1, sems1)
        return carry

    lax.fori_loop(0, NSEG, seg, 0)
    plsc.subcore_barrier()
    _copy_out(r0, acc_sh, out_hbm, c, s)


BR = 1024
GRID = NPAD // BR

_DEGP_SPEC = pl.BlockSpec((BR, DEGW), lambda i: (i, 0))
_DIS_SPEC = pl.BlockSpec((BR, 16), lambda i: (i, 0))
_ROW_SPEC = pl.BlockSpec((BR, F), lambda i: (i, 0))
_MAT_SPEC = pl.BlockSpec((F, F), lambda i: (0, 0))
_VEC_SPEC = pl.BlockSpec((1, F), lambda i: (0, 0))


def _tc1_body(degp_ref, x_ref, w1_ref, hs1_ref, dis_ref):
    deg = degp_ref[:, 0] + 1.0
    dis = lax.rsqrt(deg)[:, None]
    h = jnp.dot(x_ref[...], w1_ref[...], preferred_element_type=jnp.float32)
    hs1_ref[...] = dis * h
    dis_ref[...] = jnp.broadcast_to(dis, (BR, 16))


_tc1 = pl.pallas_call(
    _tc1_body,
    grid=(GRID,),
    in_specs=[_DEGP_SPEC, _ROW_SPEC, _MAT_SPEC],
    out_specs=[_ROW_SPEC, _DIS_SPEC],
    out_shape=[jax.ShapeDtypeStruct((N, F), jnp.float32),
               jax.ShapeDtypeStruct((NPAD, 16), jnp.float32)],
)


def _tcmid_body(dis_ref, acc_ref, hs1_ref, b1_ref, w2_ref, hs2_ref):
    dis = dis_ref[:, :1]
    z = jnp.maximum(dis * (acc_ref[...] + hs1_ref[...]) + b1_ref[...], 0.0)
    h2 = jnp.dot(z, w2_ref[...], preferred_element_type=jnp.float32)
    hs2_ref[...] = dis * h2


_tcmid = pl.pallas_call(
    _tcmid_body,
    grid=(GRID,),
    in_specs=[_DIS_SPEC, _ROW_SPEC, _ROW_SPEC, _VEC_SPEC, _MAT_SPEC],
    out_specs=_ROW_SPEC,
    out_shape=jax.ShapeDtypeStruct((N, F), jnp.float32),
)


def _tcf_body(dis_ref, acc_ref, hs2_ref, b2_ref, out_ref):
    out_ref[...] = (dis_ref[:, :1] * (acc_ref[...] + hs2_ref[...])
                    + b2_ref[...])


_tcf = pl.pallas_call(
    _tcf_body,
    grid=(GRID,),
    in_specs=[_DIS_SPEC, _ROW_SPEC, _ROW_SPEC, _VEC_SPEC],
    out_specs=_ROW_SPEC,
    out_shape=jax.ShapeDtypeStruct((N, F), jnp.float32),
)


def kernel(x, edge_index, W1, b1, W2, b2):
    src = edge_index[0].astype(jnp.int32).reshape(NS, NSEG, SCHUNK, CH)
    dst = edge_index[1].astype(jnp.int32).reshape(NS, NSEG, SCHUNK, CH)
    degp = _deg_kernel(dst)
    hs1, dis = _tc1(degp, x, W1)
    acc1 = _scatter_kernel(hs1, src, dst)
    hs2 = _tcmid(dis, acc1, hs1, b1.reshape(1, F), W2)
    acc2 = _scatter_kernel(hs2, src, dst)
    return _tcf(dis, acc2, hs2, b2.reshape(1, F))
